# Initial kernel scaffold; baseline (speedup 1.0000x reference)
import jax
import jax.numpy as jnp
from jax.experimental import pallas as pl


def kernel(x, edge_index, edge_weight, weight, bias):
    raise NotImplementedError("write your pallas kernel here")


# speedup vs baseline: 4.7667x; 4.7667x over previous
import functools

import jax
import jax.numpy as jnp
from jax import lax
from jax.experimental import pallas as pl
from jax.experimental.pallas import tpu as pltpu
from jax.experimental.pallas import tpu_sc as plsc

N = 10000
DIM = 128
NC = 16
E = 320000
MIN_NORM = 1e-15
BALL_EPS = 4e-3
MAXNORM = 1.0 - BALL_EPS

NW = 32
CHUNK = 128
CPW = 79
EPW = CPW * CHUNK
E_PAD = EPW * NW
N_PAD = 10240
ROWS_PER_TILE = N_PAD // 16

ROWS_TC = 400
GRID_TC = N // ROWS_TC


def _artanh(x):
    x = jnp.clip(x, -1.0 + 1e-7, 1.0 - 1e-7)
    return 0.5 * jnp.log((1.0 + x) / (1.0 - x))


def _row_norm(x):
    return jnp.maximum(jnp.sqrt(jnp.sum(x * x, axis=-1, keepdims=True)), MIN_NORM)


def _proj(x):
    n = _row_norm(x)
    return jnp.where(n > MAXNORM, x / n * MAXNORM, x)


def _expmap0(u):
    u_norm = _row_norm(u)
    return jnp.tanh(u_norm) * u / u_norm


def _logmap0(p):
    p_norm = _row_norm(p)
    return _artanh(p_norm) * p / p_norm


def _mobius_add(x, y):
    x2 = jnp.sum(x * x, axis=-1, keepdims=True)
    y2 = jnp.sum(y * y, axis=-1, keepdims=True)
    xy = jnp.sum(x * y, axis=-1, keepdims=True)
    num = (1.0 + 2.0 * xy + y2) * x + (1.0 - x2) * y
    denom = 1.0 + 2.0 * xy + x2 * y2
    return num / jnp.maximum(denom, MIN_NORM)


def _pre_body(x_ref, w_ref, b_ref, o_ref):
    x = x_ref[...]
    w = w_ref[...]
    b = b_ref[...]
    mx = lax.dot_general(x, w, (((1,), (1,)), ((), ())),
                         preferred_element_type=jnp.float32)
    x_norm = _row_norm(x)
    mx_norm = _row_norm(mx)
    res_c = jnp.tanh(mx_norm / x_norm * _artanh(x_norm)) * mx / mx_norm
    cond = jnp.all(mx == 0.0, axis=-1, keepdims=True)
    mv = jnp.where(cond, jnp.zeros_like(res_c), res_c)
    res = _proj(mv)
    hyp_bias = _proj(_expmap0(b))
    res = _proj(_mobius_add(res, hyp_bias))
    o_ref[...] = _logmap0(res)


_pre_call = pl.pallas_call(
    _pre_body,
    grid=(GRID_TC,),
    in_specs=[
        pl.BlockSpec((ROWS_TC, DIM), lambda i: (i, 0)),
        pl.BlockSpec((NC, DIM), lambda i: (0, 0)),
        pl.BlockSpec((1, NC), lambda i: (0, 0)),
    ],
    out_specs=pl.BlockSpec((ROWS_TC, NC), lambda i: (i, 0)),
    out_shape=jax.ShapeDtypeStruct((N, NC), jnp.float32),
)


def _post_body(p_ref, o_ref):
    s = p_ref[0] + p_ref[1]
    h = _proj(_expmap0(s))
    xt2 = _logmap0(h)
    o_ref[...] = _proj(_expmap0(xt2))


_post_call = pl.pallas_call(
    _post_body,
    grid=(GRID_TC,),
    in_specs=[pl.BlockSpec((2, ROWS_TC, NC), lambda i: (0, i, 0))],
    out_specs=pl.BlockSpec((ROWS_TC, NC), lambda i: (i, 0)),
    out_shape=jax.ShapeDtypeStruct((N, NC), jnp.float32),
)


def _sc_scatter_body(xt_hbm, src_hbm, dst_hbm, wb_hbm, out_hbm,
                src_v, dst_v, rows_v, wv_v, scaled_v, acc_sh, sem):
    cid = lax.axis_index("c")
    sid = lax.axis_index("s")
    wid = sid * 2 + cid

    zero16 = jnp.zeros((NC,), jnp.float32)
    def _zrow(r, carry):
        scaled_v[r, :] = zero16
        return carry
    lax.fori_loop(0, CHUNK, _zrow, 0)

    def _zcp(k, carry):
        pltpu.sync_copy(
            scaled_v,
            acc_sh.at[pl.ds(sid * ROWS_PER_TILE + k * CHUNK, CHUNK)])
        return carry
    lax.fori_loop(0, ROWS_PER_TILE // CHUNK, _zcp, 0)
    plsc.subcore_barrier()

    pltpu.sync_copy(src_hbm.at[wid], src_v)
    pltpu.sync_copy(dst_hbm.at[wid], dst_v)

    def _chunk(j, carry):
        pltpu.async_copy(xt_hbm.at[src_v.at[j]], rows_v, sem).wait()
        pltpu.sync_copy(wb_hbm.at[wid, j], wv_v)
        for e in range(CHUNK):
            scaled_v[e, :] = rows_v[e, :] * wv_v[e, :]
        pltpu.sync_copy(scaled_v, acc_sh.at[dst_v.at[j]], add=True)
        return carry
    lax.fori_loop(0, CPW, _chunk, 0)

    plsc.subcore_barrier()
    pltpu.sync_copy(
        acc_sh.at[pl.ds(sid * ROWS_PER_TILE, ROWS_PER_TILE)],
        out_hbm.at[cid, pl.ds(sid * ROWS_PER_TILE, ROWS_PER_TILE)])


@functools.lru_cache(maxsize=1)
def _get_sc_call():
    mesh = plsc.VectorSubcoreMesh(core_axis_name="c", subcore_axis_name="s")
    return pl.kernel(
        _sc_scatter_body,
        out_type=jax.ShapeDtypeStruct((2, N_PAD, NC), jnp.float32),
        mesh=mesh,
        compiler_params=pltpu.CompilerParams(use_tc_tiling_on_sc=False),
        scratch_types=[
            pltpu.VMEM((CPW, CHUNK), jnp.int32),
            pltpu.VMEM((CPW, CHUNK), jnp.int32),
            pltpu.VMEM((CHUNK, NC), jnp.float32),
            pltpu.VMEM((CHUNK, NC), jnp.float32),
            pltpu.VMEM((CHUNK, NC), jnp.float32),
            pltpu.VMEM_SHARED((N_PAD, NC), jnp.float32),
            pltpu.SemaphoreType.DMA,
        ],
    )


def kernel(x, edge_index, edge_weight, weight, bias):
    xt = _pre_call(x, weight, bias.reshape(1, NC))

    pad = E_PAD - E
    src = jnp.concatenate(
        [edge_index[0], jnp.zeros((pad,), jnp.int32)]).reshape(NW, CPW, CHUNK)
    dst = jnp.concatenate(
        [edge_index[1], jnp.zeros((pad,), jnp.int32)]).reshape(NW, CPW, CHUNK)
    ewp = jnp.concatenate([edge_weight, jnp.zeros((pad,), jnp.float32)])
    wb = jnp.broadcast_to(ewp[:, None], (E_PAD, NC)).reshape(NW, CPW, CHUNK, NC)

    partials = _get_sc_call()(xt, src, dst, wb)
    return _post_call(partials)


# speedup vs baseline: 9.8150x; 2.0591x over previous
import functools

import jax
import jax.numpy as jnp
from jax import lax
from jax.experimental import pallas as pl
from jax.experimental.pallas import tpu as pltpu
from jax.experimental.pallas import tpu_sc as plsc

N = 10000
DIM = 128
NC = 16
E = 320000
MIN_NORM = 1e-15
BALL_EPS = 4e-3
MAXNORM = 1.0 - BALL_EPS

NW = 32
CHUNK = 128
CPW = 79
EPW = CPW * CHUNK
E_PAD = EPW * NW
N_PAD = 10240
ROWS_PER_TILE = N_PAD // 16

ROWS_TC = 400
GRID_TC = N // ROWS_TC


def _artanh(x):
    x = jnp.clip(x, -1.0 + 1e-7, 1.0 - 1e-7)
    return 0.5 * jnp.log((1.0 + x) / (1.0 - x))


def _row_norm(x):
    return jnp.maximum(jnp.sqrt(jnp.sum(x * x, axis=-1, keepdims=True)), MIN_NORM)


def _proj(x):
    n = _row_norm(x)
    return jnp.where(n > MAXNORM, x / n * MAXNORM, x)


def _expmap0(u):
    u_norm = _row_norm(u)
    return jnp.tanh(u_norm) * u / u_norm


def _logmap0(p):
    p_norm = _row_norm(p)
    return _artanh(p_norm) * p / p_norm


def _mobius_add(x, y):
    x2 = jnp.sum(x * x, axis=-1, keepdims=True)
    y2 = jnp.sum(y * y, axis=-1, keepdims=True)
    xy = jnp.sum(x * y, axis=-1, keepdims=True)
    num = (1.0 + 2.0 * xy + y2) * x + (1.0 - x2) * y
    denom = 1.0 + 2.0 * xy + x2 * y2
    return num / jnp.maximum(denom, MIN_NORM)


def _pre_body(x_ref, w_ref, b_ref, o_ref):
    x = x_ref[...]
    w = w_ref[...]
    b = b_ref[...]
    mx = lax.dot_general(x, w, (((1,), (1,)), ((), ())),
                         preferred_element_type=jnp.float32)
    x_norm = _row_norm(x)
    mx_norm = _row_norm(mx)
    res_c = jnp.tanh(mx_norm / x_norm * _artanh(x_norm)) * mx / mx_norm
    cond = jnp.all(mx == 0.0, axis=-1, keepdims=True)
    mv = jnp.where(cond, jnp.zeros_like(res_c), res_c)
    res = _proj(mv)
    hyp_bias = _proj(_expmap0(b))
    res = _proj(_mobius_add(res, hyp_bias))
    o_ref[...] = _logmap0(res)


_pre_call = pl.pallas_call(
    _pre_body,
    grid=(GRID_TC,),
    in_specs=[
        pl.BlockSpec((ROWS_TC, DIM), lambda i: (i, 0)),
        pl.BlockSpec((NC, DIM), lambda i: (0, 0)),
        pl.BlockSpec((1, NC), lambda i: (0, 0)),
    ],
    out_specs=pl.BlockSpec((ROWS_TC, NC), lambda i: (i, 0)),
    out_shape=jax.ShapeDtypeStruct((N, NC), jnp.float32),
)


def _post_body(p_ref, o_ref):
    s = p_ref[0] + p_ref[1]
    h = _proj(_expmap0(s))
    xt2 = _logmap0(h)
    o_ref[...] = _proj(_expmap0(xt2))


_post_call = pl.pallas_call(
    _post_body,
    grid=(GRID_TC,),
    in_specs=[pl.BlockSpec((2, ROWS_TC, NC), lambda i: (0, i, 0))],
    out_specs=pl.BlockSpec((ROWS_TC, NC), lambda i: (i, 0)),
    out_shape=jax.ShapeDtypeStruct((N, NC), jnp.float32),
)


def _sc_scatter_body(xt_hbm, src_hbm, dst_hbm, wb_hbm, out_hbm,
                src_v, dst_v, rows_v, wv_v, scaled_v, acc_sh, sem):
    cid = lax.axis_index("c")
    sid = lax.axis_index("s")
    wid = sid * 2 + cid

    zero16 = jnp.zeros((NC,), jnp.float32)
    def _zrow(r, carry):
        scaled_v[r, :] = zero16
        return carry
    lax.fori_loop(0, CHUNK, _zrow, 0)

    def _zcp(k, carry):
        pltpu.sync_copy(
            scaled_v,
            acc_sh.at[pl.ds(sid * ROWS_PER_TILE + k * CHUNK, CHUNK)])
        return carry
    lax.fori_loop(0, ROWS_PER_TILE // CHUNK, _zcp, 0)
    plsc.subcore_barrier()

    pltpu.sync_copy(src_hbm.at[wid], src_v)
    pltpu.sync_copy(dst_hbm.at[wid], dst_v)
    pltpu.sync_copy(wb_hbm.at[wid], wv_v)

    def _chunk(j, carry):
        pltpu.async_copy(xt_hbm.at[src_v.at[j]], rows_v, sem).wait()
        for g in range(CHUNK // NC):
            w16 = wv_v[j, pl.ds(g * NC, NC)]
            for e in range(NC):
                wbc = lax.gather(
                    w16, jnp.full((NC, 1), e, jnp.int32),
                    lax.GatherDimensionNumbers(
                        offset_dims=(), collapsed_slice_dims=(0,),
                        start_index_map=(0,)),
                    slice_sizes=(1,),
                    mode=lax.GatherScatterMode.PROMISE_IN_BOUNDS)
                scaled_v[g * NC + e, :] = rows_v[g * NC + e, :] * wbc
        pltpu.sync_copy(scaled_v, acc_sh.at[dst_v.at[j]], add=True)
        return carry
    lax.fori_loop(0, CPW, _chunk, 0)

    plsc.subcore_barrier()
    pltpu.sync_copy(
        acc_sh.at[pl.ds(sid * ROWS_PER_TILE, ROWS_PER_TILE)],
        out_hbm.at[cid, pl.ds(sid * ROWS_PER_TILE, ROWS_PER_TILE)])


@functools.lru_cache(maxsize=1)
def _get_sc_call():
    mesh = plsc.VectorSubcoreMesh(core_axis_name="c", subcore_axis_name="s")
    return pl.kernel(
        _sc_scatter_body,
        out_type=jax.ShapeDtypeStruct((2, N_PAD, NC), jnp.float32),
        mesh=mesh,
        compiler_params=pltpu.CompilerParams(use_tc_tiling_on_sc=False),
        scratch_types=[
            pltpu.VMEM((CPW, CHUNK), jnp.int32),
            pltpu.VMEM((CPW, CHUNK), jnp.int32),
            pltpu.VMEM((CHUNK, NC), jnp.float32),
            pltpu.VMEM((CPW, CHUNK), jnp.float32),
            pltpu.VMEM((CHUNK, NC), jnp.float32),
            pltpu.VMEM_SHARED((N_PAD, NC), jnp.float32),
            pltpu.SemaphoreType.DMA,
        ],
    )


def kernel(x, edge_index, edge_weight, weight, bias):
    xt = _pre_call(x, weight, bias.reshape(1, NC))

    pad = E_PAD - E
    src = jnp.concatenate(
        [edge_index[0], jnp.zeros((pad,), jnp.int32)]).reshape(NW, CPW, CHUNK)
    dst = jnp.concatenate(
        [edge_index[1], jnp.zeros((pad,), jnp.int32)]).reshape(NW, CPW, CHUNK)
    ewp = jnp.concatenate(
        [edge_weight, jnp.zeros((pad,), jnp.float32)]).reshape(NW, CPW, CHUNK)

    partials = _get_sc_call()(xt, src, dst, ewp)
    return _post_call(partials)


# speedup vs baseline: 11.5351x; 1.1753x over previous
import functools

import jax
import jax.numpy as jnp
from jax import lax
from jax.experimental import pallas as pl
from jax.experimental.pallas import tpu as pltpu
from jax.experimental.pallas import tpu_sc as plsc

N = 10000
DIM = 128
NC = 16
E = 320000
MIN_NORM = 1e-15
BALL_EPS = 4e-3
MAXNORM = 1.0 - BALL_EPS

NW = 32
CHUNK = 128
CPW = 80
EPW = CPW * CHUNK
E_PAD = EPW * NW
N_PAD = 10240
ROWS_PER_TILE = N_PAD // 16

ROWS_TC = 400
GRID_TC = N // ROWS_TC


def _artanh(x):
    x = jnp.clip(x, -1.0 + 1e-7, 1.0 - 1e-7)
    return 0.5 * jnp.log((1.0 + x) / (1.0 - x))


def _row_norm(x):
    return jnp.maximum(jnp.sqrt(jnp.sum(x * x, axis=-1, keepdims=True)), MIN_NORM)


def _proj(x):
    n = _row_norm(x)
    return jnp.where(n > MAXNORM, x / n * MAXNORM, x)


def _expmap0(u):
    u_norm = _row_norm(u)
    return jnp.tanh(u_norm) * u / u_norm


def _logmap0(p):
    p_norm = _row_norm(p)
    return _artanh(p_norm) * p / p_norm


def _mobius_add(x, y):
    x2 = jnp.sum(x * x, axis=-1, keepdims=True)
    y2 = jnp.sum(y * y, axis=-1, keepdims=True)
    xy = jnp.sum(x * y, axis=-1, keepdims=True)
    num = (1.0 + 2.0 * xy + y2) * x + (1.0 - x2) * y
    denom = 1.0 + 2.0 * xy + x2 * y2
    return num / jnp.maximum(denom, MIN_NORM)


def _pre_body(x_ref, w_ref, b_ref, o_ref):
    x = x_ref[...]
    w = w_ref[...]
    b = b_ref[...]
    mx = lax.dot_general(x, w, (((1,), (1,)), ((), ())),
                         preferred_element_type=jnp.float32)
    x_norm = _row_norm(x)
    mx_norm = _row_norm(mx)
    res_c = jnp.tanh(mx_norm / x_norm * _artanh(x_norm)) * mx / mx_norm
    cond = jnp.all(mx == 0.0, axis=-1, keepdims=True)
    mv = jnp.where(cond, jnp.zeros_like(res_c), res_c)
    res = _proj(mv)
    hyp_bias = _proj(_expmap0(b))
    res = _proj(_mobius_add(res, hyp_bias))
    o_ref[...] = _logmap0(res)


_pre_call = pl.pallas_call(
    _pre_body,
    grid=(GRID_TC,),
    in_specs=[
        pl.BlockSpec((ROWS_TC, DIM), lambda i: (i, 0)),
        pl.BlockSpec((NC, DIM), lambda i: (0, 0)),
        pl.BlockSpec((1, NC), lambda i: (0, 0)),
    ],
    out_specs=pl.BlockSpec((ROWS_TC, NC), lambda i: (i, 0)),
    out_shape=jax.ShapeDtypeStruct((N, NC), jnp.float32),
)


def _post_body(p_ref, o_ref):
    s = p_ref[0] + p_ref[1]
    h = _proj(_expmap0(s))
    xt2 = _logmap0(h)
    o_ref[...] = _proj(_expmap0(xt2))


_post_call = pl.pallas_call(
    _post_body,
    grid=(GRID_TC,),
    in_specs=[pl.BlockSpec((2, ROWS_TC, NC), lambda i: (0, i, 0))],
    out_specs=pl.BlockSpec((ROWS_TC, NC), lambda i: (i, 0)),
    out_shape=jax.ShapeDtypeStruct((N, NC), jnp.float32),
)


def _sc_scatter_body(xt_hbm, src_hbm, dst_hbm, wb_hbm, out_hbm,
                     src_v, dst_v, wv_v, rows0, rows1, scaled_v, acc_sh,
                     g0, g1):
    cid = lax.axis_index("c")
    sid = lax.axis_index("s")
    wid = sid * 2 + cid

    zero16 = jnp.zeros((NC,), jnp.float32)
    def _zrow(r, carry):
        scaled_v[r, :] = zero16
        return carry
    lax.fori_loop(0, CHUNK, _zrow, 0)

    def _zcp(k, carry):
        pltpu.sync_copy(
            scaled_v,
            acc_sh.at[pl.ds(sid * ROWS_PER_TILE + k * CHUNK, CHUNK)])
        return carry
    lax.fori_loop(0, ROWS_PER_TILE // CHUNK, _zcp, 0)
    plsc.subcore_barrier()

    pltpu.sync_copy(src_hbm.at[wid], src_v)
    pltpu.sync_copy(dst_hbm.at[wid], dst_v)
    pltpu.sync_copy(wb_hbm.at[wid], wv_v)

    def _scale_and_scatter(j, rows_v):
        for g in range(CHUNK // NC):
            w16 = wv_v[j, pl.ds(g * NC, NC)]
            for e in range(NC):
                wbc = lax.gather(
                    w16, jnp.full((NC, 1), e, jnp.int32),
                    lax.GatherDimensionNumbers(
                        offset_dims=(), collapsed_slice_dims=(0,),
                        start_index_map=(0,)),
                    slice_sizes=(1,),
                    mode=lax.GatherScatterMode.PROMISE_IN_BOUNDS)
                scaled_v[g * NC + e, :] = rows_v[g * NC + e, :] * wbc
        pltpu.sync_copy(scaled_v, acc_sh.at[dst_v.at[j]], add=True)

    pltpu.async_copy(xt_hbm.at[src_v.at[0]], rows0, g0)

    def _pair(i, carry):
        j0 = 2 * i
        j1 = j0 + 1
        pltpu.async_copy(xt_hbm.at[src_v.at[j1]], rows1, g1)
        pltpu.make_async_copy(xt_hbm.at[src_v.at[j0]], rows0, g0).wait()
        _scale_and_scatter(j0, rows0)

        @pl.when(i < CPW // 2 - 1)
        def _prefetch():
            pltpu.async_copy(xt_hbm.at[src_v.at[j0 + 2]], rows0, g0)

        pltpu.make_async_copy(xt_hbm.at[src_v.at[j1]], rows1, g1).wait()
        _scale_and_scatter(j1, rows1)
        return carry
    lax.fori_loop(0, CPW // 2, _pair, 0)

    plsc.subcore_barrier()
    pltpu.sync_copy(
        acc_sh.at[pl.ds(sid * ROWS_PER_TILE, ROWS_PER_TILE)],
        out_hbm.at[cid, pl.ds(sid * ROWS_PER_TILE, ROWS_PER_TILE)])


@functools.lru_cache(maxsize=1)
def _get_sc_call():
    mesh = plsc.VectorSubcoreMesh(core_axis_name="c", subcore_axis_name="s")
    return pl.kernel(
        _sc_scatter_body,
        out_type=jax.ShapeDtypeStruct((2, N_PAD, NC), jnp.float32),
        mesh=mesh,
        compiler_params=pltpu.CompilerParams(use_tc_tiling_on_sc=False),
        scratch_types=[
            pltpu.VMEM((CPW, CHUNK), jnp.int32),
            pltpu.VMEM((CPW, CHUNK), jnp.int32),
            pltpu.VMEM((CPW, CHUNK), jnp.float32),
            pltpu.VMEM((CHUNK, NC), jnp.float32),
            pltpu.VMEM((CHUNK, NC), jnp.float32),
            pltpu.VMEM((CHUNK, NC), jnp.float32),
            pltpu.VMEM_SHARED((N_PAD, NC), jnp.float32),
            pltpu.SemaphoreType.DMA,
            pltpu.SemaphoreType.DMA,
        ],
    )


def kernel(x, edge_index, edge_weight, weight, bias):
    xt = _pre_call(x, weight, bias.reshape(1, NC))

    pad = E_PAD - E
    src = jnp.concatenate(
        [edge_index[0], jnp.zeros((pad,), jnp.int32)]).reshape(NW, CPW, CHUNK)
    dst = jnp.concatenate(
        [edge_index[1], jnp.zeros((pad,), jnp.int32)]).reshape(NW, CPW, CHUNK)
    ewp = jnp.concatenate(
        [edge_weight, jnp.zeros((pad,), jnp.float32)]).reshape(NW, CPW, CHUNK)

    partials = _get_sc_call()(xt, src, dst, ewp)
    return _post_call(partials)
